# Initial kernel scaffold; baseline (speedup 1.0000x reference)
import jax
import jax.numpy as jnp
from jax.experimental import pallas as pl


def kernel(inputs, weight):
    raise NotImplementedError("write your pallas kernel here")


# speedup vs baseline: 3.4428x; 3.4428x over previous
import jax
import jax.numpy as jnp
from jax.experimental import pallas as pl
from jax.experimental.pallas import tpu as pltpu
from jax.experimental.pallas import tpu_sc as plsc

N_TOK = 65536
N_EMB = 1024
DIM = 64
BLK = 1024
GW = 128


def _tc_body(x_ref, w_ref, idx_ref, loss_ref, acc_ref):
    i = pl.program_id(0)
    x = x_ref[...]
    w = w_ref[...]
    c = jax.lax.dot_general(x, w, (((1,), (1,)), ((), ())),
                            preferred_element_type=jnp.float32)
    a = jnp.sum(x * x, axis=1, keepdims=True)
    b = jnp.sum(w * w, axis=1)[None, :]
    dist = (a + b) - 2.0 * c
    m = jnp.min(dist, axis=1, keepdims=True)
    jidx = jax.lax.broadcasted_iota(jnp.int32, dist.shape, 1)
    idx = jnp.min(jnp.where(dist == m, jidx, N_EMB), axis=1)
    idx_ref[...] = idx.reshape(BLK // 128, 128)

    @pl.when(i == 0)
    def _():
        acc_ref[0] = 0.0

    acc_ref[0] += jnp.sum(m)

    @pl.when(i == pl.num_programs(0) - 1)
    def _():
        loss_ref[...] = jnp.full((1, 1), acc_ref[0] * (1.25 / (N_TOK * DIM)),
                                 dtype=jnp.float32)


def _tc_argmin_loss(inputs, weight):
    return pl.pallas_call(
        _tc_body,
        grid=(N_TOK // BLK,),
        in_specs=[
            pl.BlockSpec((BLK, DIM), lambda i: (i, 0)),
            pl.BlockSpec((N_EMB, DIM), lambda i: (0, 0)),
        ],
        out_specs=[
            pl.BlockSpec((BLK // 128, 128), lambda i: (i, 0)),
            pl.BlockSpec((1, 1), lambda i: (0, 0)),
        ],
        out_shape=[
            jax.ShapeDtypeStruct((N_TOK // 128, 128), jnp.int32),
            jax.ShapeDtypeStruct((1, 1), jnp.float32),
        ],
        scratch_shapes=[pltpu.SMEM((1,), jnp.float32)],
    )(inputs, weight)


SC_NC = 2
SC_NS = 16
SC_NW = SC_NC * SC_NS
SC_CH = 128
ROWS_PER_W = N_TOK // SC_NW
N_CH = ROWS_PER_W // SC_CH


def _sc_gather(w_pad, idx):
    mesh = plsc.VectorSubcoreMesh(core_axis_name="c", subcore_axis_name="s")

    @pl.kernel(out_type=jax.ShapeDtypeStruct((N_TOK, 128), jnp.float32),
               mesh=mesh,
               scratch_types=[
                   pltpu.VMEM((SC_CH,), jnp.int32),
                   pltpu.VMEM((SC_CH, 128), jnp.float32),
                   pltpu.SemaphoreType.DMA,
               ])
    def k(w_hbm, i_hbm, o_hbm, idx_v, rows_v, sem):
        wid = jax.lax.axis_index("s") * SC_NC + jax.lax.axis_index("c")
        base = wid * ROWS_PER_W

        @pl.loop(0, N_CH)
        def _(c):
            off = base + c * SC_CH
            pltpu.sync_copy(i_hbm.at[pl.ds(off, SC_CH)], idx_v)
            pltpu.async_copy(w_hbm.at[idx_v], rows_v, sem).wait()
            pltpu.sync_copy(rows_v, o_hbm.at[pl.ds(off, SC_CH)])

    return k(w_pad, idx)


def kernel(inputs, weight):
    idx2d, loss2d = _tc_argmin_loss(inputs, weight)
    idx = idx2d.reshape(N_TOK)
    w_pad = jnp.concatenate(
        [weight, jnp.zeros((N_EMB, 128 - DIM), jnp.float32)], axis=1)
    quantized = _sc_gather(w_pad, idx)[:, :DIM]
    return loss2d[0, 0], quantized, idx
